# baseline (speedup 1.0000x reference)
import functools

import jax
import jax.numpy as jnp
import numpy as np
from jax import lax
from jax.experimental import pallas as pl
from jax.experimental.pallas import tpu as pltpu
from jax.experimental.pallas import tpu_sc as plsc

NUM_TOKENS = 16384
INPUT_DIM = 2048
NUM_EXPERTS = 64
BLOCK_T = 1024
N_PANELS = NUM_TOKENS // BLOCK_T

NUM_CORES = 2
NUM_SUBCORES = 16
NUM_WORKERS = NUM_CORES * NUM_SUBCORES
ROWS_PER_W = NUM_TOKENS // NUM_WORKERS
GROUPS_PER_W = ROWS_PER_W // 16
_CHUNK = ROWS_PER_W * NUM_EXPERTS

_KS0 = np.uint32(0)
_KS1 = np.uint32(1)
_KS2 = np.uint32(0 ^ 1 ^ 0x1BD11BDA)
_ROT_A = (13, 15, 26, 6)
_ROT_B = (17, 29, 16, 24)


def _rotl(x, r):
    return (x << np.uint32(r)) | (x >> np.uint32(32 - r))


def _rounds(x0, x1, rots):
    for r in rots:
        x0 = x0 + x1
        x1 = _rotl(x1, r)
        x1 = x0 ^ x1
    return x0, x1


def _noise_from_counter(c_lo):
    x0 = jnp.zeros(c_lo.shape, jnp.uint32) + _KS0
    x1 = c_lo + _KS1
    x0, x1 = _rounds(x0, x1, _ROT_A)
    x0 = x0 + _KS1
    x1 = x1 + _KS2 + np.uint32(1)
    x0, x1 = _rounds(x0, x1, _ROT_B)
    x0 = x0 + _KS2
    x1 = x1 + _KS0 + np.uint32(2)
    x0, x1 = _rounds(x0, x1, _ROT_A)
    x0 = x0 + _KS0
    x1 = x1 + _KS1 + np.uint32(3)
    x0, x1 = _rounds(x0, x1, _ROT_B)
    x0 = x0 + _KS1
    x1 = x1 + _KS2 + np.uint32(4)
    x0, x1 = _rounds(x0, x1, _ROT_A)
    x0 = x0 + _KS2
    x1 = x1 + _KS0 + np.uint32(5)
    bits = x0 ^ x1
    fb = (bits >> np.uint32(9)) | np.uint32(0x3F800000)
    f = lax.bitcast_convert_type(fb, jnp.float32) - np.float32(1.0)
    lo = np.nextafter(np.float32(-1.0), np.float32(0.0), dtype=np.float32)
    hi = np.float32(1.0)
    u = lax.max(jnp.float32(lo), f * (hi - lo) + lo)
    return np.float32(np.sqrt(2.0)) * lax.erf_inv(u)


def _logits_body(x_ref, w_ref, b_ref, o_ref):
    logits_t = lax.dot_general(
        w_ref[...], x_ref[...],
        dimension_numbers=(((1,), (1,)), ((), ())),
        preferred_element_type=jnp.float32,
    )
    i = pl.program_id(0)
    c_lo = (jnp.uint32(i * (BLOCK_T * NUM_EXPERTS))
            + lax.broadcasted_iota(jnp.uint32, logits_t.shape, 1)
            * np.uint32(NUM_EXPERTS)
            + lax.broadcasted_iota(jnp.uint32, logits_t.shape, 0))
    noisy_t = logits_t + b_ref[...] + _noise_from_counter(c_lo)
    o_ref[...] = noisy_t.reshape(1, NUM_EXPERTS, BLOCK_T)


def _noisy_logits_panels(x, W, b):
    n_tokens = x.shape[0]
    grid = (n_tokens // BLOCK_T,)
    return pl.pallas_call(
        _logits_body,
        grid=grid,
        in_specs=[
            pl.BlockSpec((BLOCK_T, INPUT_DIM), lambda i: (i, 0)),
            pl.BlockSpec((NUM_EXPERTS, INPUT_DIM), lambda i: (0, 0)),
            pl.BlockSpec((NUM_EXPERTS, 1), lambda i: (0, 0)),
        ],
        out_specs=pl.BlockSpec((1, NUM_EXPERTS, BLOCK_T), lambda i: (i, 0, 0)),
        out_shape=jax.ShapeDtypeStruct((N_PANELS, NUM_EXPERTS, BLOCK_T),
                                       jnp.float32),
    )(x, W, b.reshape(NUM_EXPERTS, 1))


_SC_MESH = plsc.VectorSubcoreMesh(
    core_axis_name="c", subcore_axis_name="s",
    num_cores=NUM_CORES, num_subcores=NUM_SUBCORES)

_N_STREAMS = 4
_E_PER_STREAM = NUM_EXPERTS // _N_STREAMS
_W_PER_PANEL = BLOCK_T // ROWS_PER_W


def _merge_top2(a, b):
    ma1, ia1, ma2, ia2 = a
    mb1, ib1, mb2, ib2 = b
    c = mb1 > ma1
    ca = mb1 > ma2
    m2a = jnp.where(ca, mb1, ma2)
    i2a = jnp.where(ca, ib1, ia2)
    cb = mb2 > ma1
    m2b = jnp.where(cb, mb2, ma1)
    i2b = jnp.where(cb, ib2, ia1)
    m1 = jnp.where(c, mb1, ma1)
    i1 = jnp.where(c, ib1, ia1)
    m2 = jnp.where(c, m2b, m2a)
    i2 = jnp.where(c, i2b, i2a)
    return m1, i1, m2, i2


@functools.partial(
    pl.kernel,
    out_type=jax.ShapeDtypeStruct((NUM_TOKENS * NUM_EXPERTS,), jnp.float32),
    mesh=_SC_MESH,
    scratch_types=[
        pltpu.VMEM((NUM_EXPERTS, ROWS_PER_W), jnp.float32),
        pltpu.VMEM((_CHUNK,), jnp.float32),
    ],
    compiler_params=pltpu.CompilerParams(needs_layout_passes=False),
)
def _sc_route(logits_p_hbm, out_hbm, lbuf, obuf):
    wid = lax.axis_index("s") * NUM_CORES + lax.axis_index("c")
    panel = wid // _W_PER_PANEL
    off = (wid % _W_PER_PANEL) * ROWS_PER_W
    pltpu.sync_copy(logits_p_hbm.at[panel, :, pl.ds(off, ROWS_PER_W)], lbuf)

    lanes = lax.iota(jnp.int32, 16)
    neg_inf = jnp.full((16,), -jnp.inf, jnp.float32)
    zeros = jnp.zeros((16,), jnp.float32)
    zero_i = jnp.zeros((16,), jnp.int32)

    def group_body(j, _):
        r0 = j * 16
        gbase = r0 * NUM_EXPERTS
        for k in range(NUM_EXPERTS):
            obuf[pl.ds(gbase + k * 16, 16)] = zeros

        states = []
        for q in range(_N_STREAMS):
            m1, i1, m2, i2 = neg_inf, zero_i, neg_inf, zero_i
            for t in range(_E_PER_STREAM):
                e = q * _E_PER_STREAM + t
                v = lbuf[e, pl.ds(r0, 16)]
                ei = jnp.full((16,), e, jnp.int32)
                gt1 = v > m1
                gt2 = v > m2
                m2 = jnp.where(gt2, v, m2)
                i2 = jnp.where(gt2, ei, i2)
                m2 = jnp.where(gt1, m1, m2)
                i2 = jnp.where(gt1, i1, i2)
                m1 = jnp.where(gt1, v, m1)
                i1 = jnp.where(gt1, ei, i1)
            states.append((m1, i1, m2, i2))
        s01 = _merge_top2(states[0], states[1])
        s23 = _merge_top2(states[2], states[3])
        m1, i1, m2, i2 = _merge_top2(s01, s23)

        t = jnp.exp(m2 - m1)
        p1 = 1.0 / (1.0 + t)
        p2 = t * p1
        rbase = (r0 + lanes) * NUM_EXPERTS
        plsc.store_scatter(obuf, [rbase + i1], p1)
        plsc.store_scatter(obuf, [rbase + i2], p2)
        return 0

    lax.fori_loop(0, GROUPS_PER_W, group_body, 0)
    pltpu.sync_copy(obuf, out_hbm.at[pl.ds(wid * _CHUNK, _CHUNK)])


@jax.jit
def kernel(x, W, b):
    n_tokens = x.shape[0]
    noisy_p = _noisy_logits_panels(x, W, b)
    flat = _sc_route(noisy_p)
    return flat.reshape(n_tokens, NUM_EXPERTS)


# speedup vs baseline: 1.6323x; 1.6323x over previous
import jax
import jax.numpy as jnp
import numpy as np
from jax import lax
from jax.experimental import pallas as pl

NUM_TOKENS = 16384
INPUT_DIM = 2048
NUM_EXPERTS = 64
BLOCK_T = 1024

_KS0 = np.uint32(0)
_KS1 = np.uint32(1)
_KS2 = np.uint32(0 ^ 1 ^ 0x1BD11BDA)
_ROT_A = (13, 15, 26, 6)
_ROT_B = (17, 29, 16, 24)


def _rotl(x, r):
    return (x << np.uint32(r)) | (x >> np.uint32(32 - r))


def _rounds(x0, x1, rots):
    for r in rots:
        x0 = x0 + x1
        x1 = _rotl(x1, r)
        x1 = x0 ^ x1
    return x0, x1


def _noise_block(flat_base, shape):
    c_lo = (jnp.uint32(flat_base)
            + lax.broadcasted_iota(jnp.uint32, shape, 0) * np.uint32(shape[1])
            + lax.broadcasted_iota(jnp.uint32, shape, 1))
    x0 = jnp.zeros(shape, jnp.uint32) + _KS0
    x1 = c_lo + _KS1
    x0, x1 = _rounds(x0, x1, _ROT_A)
    x0 = x0 + _KS1
    x1 = x1 + _KS2 + np.uint32(1)
    x0, x1 = _rounds(x0, x1, _ROT_B)
    x0 = x0 + _KS2
    x1 = x1 + _KS0 + np.uint32(2)
    x0, x1 = _rounds(x0, x1, _ROT_A)
    x0 = x0 + _KS0
    x1 = x1 + _KS1 + np.uint32(3)
    x0, x1 = _rounds(x0, x1, _ROT_B)
    x0 = x0 + _KS1
    x1 = x1 + _KS2 + np.uint32(4)
    x0, x1 = _rounds(x0, x1, _ROT_A)
    x0 = x0 + _KS2
    x1 = x1 + _KS0 + np.uint32(5)
    bits = x0 ^ x1
    fb = (bits >> np.uint32(9)) | np.uint32(0x3F800000)
    f = lax.bitcast_convert_type(fb, jnp.float32) - np.float32(1.0)
    lo = np.nextafter(np.float32(-1.0), np.float32(0.0), dtype=np.float32)
    hi = np.float32(1.0)
    u = lax.max(jnp.float32(lo), f * (hi - lo) + lo)
    return np.float32(np.sqrt(2.0)) * lax.erf_inv(u)


def _gating_body(x_ref, w_ref, b_ref, n_ref, o_ref):
    logits = lax.dot_general(
        x_ref[...], w_ref[...],
        dimension_numbers=(((1,), (1,)), ((), ())),
        preferred_element_type=jnp.float32,
    )
    noisy = logits + b_ref[...] + n_ref[...]

    col = lax.broadcasted_iota(jnp.int32, noisy.shape, 1)
    m1 = jnp.max(noisy, axis=-1, keepdims=True)
    i1 = jnp.min(jnp.where(noisy == m1, col, NUM_EXPERTS), axis=-1,
                 keepdims=True)
    is1 = col == i1
    masked = jnp.where(is1, -jnp.inf, noisy)
    m2 = jnp.max(masked, axis=-1, keepdims=True)
    i2 = jnp.min(jnp.where(masked == m2, col, NUM_EXPERTS), axis=-1,
                 keepdims=True)
    is2 = col == i2

    t = jnp.exp(m2 - m1)
    p1 = 1.0 / (1.0 + t)
    o_ref[...] = jnp.where(is1, p1, 0.0) + jnp.where(is2, t * p1, 0.0)


_NOISE = _noise_block(0, (NUM_TOKENS, NUM_EXPERTS))


@jax.jit
def kernel(x, W, b):
    n_tokens = x.shape[0]
    grid = (n_tokens // BLOCK_T,)
    return pl.pallas_call(
        _gating_body,
        grid=grid,
        in_specs=[
            pl.BlockSpec((BLOCK_T, INPUT_DIM), lambda i: (i, 0)),
            pl.BlockSpec((NUM_EXPERTS, INPUT_DIM), lambda i: (0, 0)),
            pl.BlockSpec((1, NUM_EXPERTS), lambda i: (0, 0)),
            pl.BlockSpec((BLOCK_T, NUM_EXPERTS), lambda i: (i, 0)),
        ],
        out_specs=pl.BlockSpec((BLOCK_T, NUM_EXPERTS), lambda i: (i, 0)),
        out_shape=jax.ShapeDtypeStruct((n_tokens, NUM_EXPERTS), jnp.float32),
    )(x, W, b.reshape(1, NUM_EXPERTS), _NOISE)
